# baseline (speedup 1.0000x reference)
import dataclasses
import functools

import numpy as np

import jax
import jax.numpy as jnp
from jax import lax
from jax.experimental import pallas as pl
from jax.experimental.pallas import tpu as pltpu
from jax.experimental.pallas import tpu_sc as plsc

OUT = 7
SR = 2
G = OUT * SR
IMG = 1024.0
EPS = 1e-6

_SIZES = (256, 128, 64, 32)
_BASES = (0, 65536, 81920, 86016)
_NROWS = 87040

_T1 = (224.0 * 0.5) ** 2 * 2.0 ** (-2 * EPS)
_T2 = 224.0**2 * 2.0 ** (-2 * EPS)
_T3 = (224.0 * 2.0) ** 2 * 2.0 ** (-2 * EPS)

NC = 2
NS = 16
NW = NC * NS
KPAD = 1024
BPW = KPAD // NW
ACC = OUT * OUT * 256
GP = 16
ACCP = 8 * OUT * 256

_PERM = np.array(
    [g * 32 + (i // 2 if i % 2 == 0 else 16 + i // 2)
     for g in range(8) for i in range(32)], dtype=np.int32)


def _sc_kernel(table_hbm, boxesT_hbm, out_hbm,
               boxes_v, x1s, y1s, bws, bhs, basei, wfi,
               idxbuf, rows, acc,
               sem0, sem1, sem2, sem3, semo0, semo1):
    wid = lax.axis_index("s") * NC + lax.axis_index("c")
    base_box = wid * BPW

    for i in range(4):
        pltpu.sync_copy(boxesT_hbm.at[i, pl.ds(base_box, BPW)], boxes_v.at[i])

    for g in range(BPW // 16):
        sl = pl.ds(g * 16, 16)
        x1 = boxes_v[0, sl]
        y1 = boxes_v[1, sl]
        x2 = boxes_v[2, sl]
        y2 = boxes_v[3, sl]
        area = (x2 - x1) * (y2 - y1)
        i32 = jnp.int32
        one = jnp.ones((16,), i32)
        zero = jnp.zeros((16,), i32)
        cnt = (jnp.where(area >= _T1, one, zero)
               + jnp.where(area >= _T2, one, zero)
               + jnp.where(area >= _T3, one, zero))
        scale = jnp.where(cnt == 0, 0.25,
                          jnp.where(cnt == 1, 0.125,
                                    jnp.where(cnt == 2, 0.0625, 0.03125)))
        wi = jnp.where(cnt == 0, _SIZES[0],
                       jnp.where(cnt == 1, _SIZES[1],
                                 jnp.where(cnt == 2, _SIZES[2], _SIZES[3])))
        bi = jnp.where(cnt == 0, _BASES[0],
                       jnp.where(cnt == 1, _BASES[1],
                                 jnp.where(cnt == 2, _BASES[2], _BASES[3])))
        x1f = x1 * scale
        y1f = y1 * scale
        rw = jnp.maximum(x2 * scale - x1f, 1.0)
        rh = jnp.maximum(y2 * scale - y1f, 1.0)
        x1s[sl] = x1f
        y1s[sl] = y1f
        bws[sl] = rw / float(OUT)
        bhs[sl] = rh / float(OUT)
        basei[sl] = bi.astype(i32)
        wfi[sl] = wi.astype(i32)

    lane = lax.iota(jnp.int32, 16)
    t = (lane.astype(jnp.float32) + 0.5) * (1.0 / SR)
    act = lane < G

    sems = (sem0, sem1, sem2, sem3)
    ioq = lane >> 2
    dymask = ((lane >> 1) & 1) == 1
    dxv = lane & 1

    @pl.loop(0, BPW)
    def _box(b):
        par_even = (b % 2) == 0
        par_off = (b % 2) * ACCP

        @pl.when(jnp.logical_and(b >= 2, par_even))
        def _():
            pltpu.make_async_copy(acc.at[pl.ds(0, ACC)],
                                  out_hbm.at[base_box + b], semo0).wait()

        @pl.when(jnp.logical_and(b >= 2, jnp.logical_not(par_even)))
        def _():
            pltpu.make_async_copy(acc.at[pl.ds(ACCP, ACC)],
                                  out_hbm.at[base_box + b], semo1).wait()

        x1b = x1s[pl.ds(b, 16)][0]
        y1b = y1s[pl.ds(b, 16)][0]
        bwb = bws[pl.ds(b, 16)][0]
        bhb = bhs[pl.ds(b, 16)][0]
        bb = basei[pl.ds(b, 16)][0]
        wib = wfi[pl.ds(b, 16)][0]
        wfb = wib.astype(jnp.float32)

        vx = jnp.clip(x1b + t * bwb, 0.0, wfb - 1.0)
        xli = jnp.minimum(vx.astype(jnp.int32), wib - 2)
        fx = vx - xli.astype(jnp.float32)
        wx0 = jnp.where(act, (1.0 - fx) * 0.25, 0.0)
        wx1 = jnp.where(act, fx * 0.25, 0.0)

        vy = jnp.clip(y1b + t * bhb, 0.0, wfb - 1.0)
        yli = jnp.minimum(vy.astype(jnp.int32), wib - 2)
        fy = vy - yli.astype(jnp.float32)
        row0 = bb + yli * wib
        row1 = row0 + wib
        wy0 = 1.0 - fy
        wy1 = fy

        def build_start(gy, slot, sem):
            gyv = jnp.full((16,), gy, jnp.int32)
            r0 = row0.at[gyv].get(mode="promise_in_bounds")
            r1 = row1.at[gyv].get(mode="promise_in_bounds")
            rsel = jnp.where(dymask, r1, r0) + dxv
            for k in range(4):
                gxk = ioq + 4 * k
                xk = xli.at[gxk].get(mode="promise_in_bounds")
                idxbuf[slot, pl.ds(16 * k, 16)] = xk + rsel
            pltpu.async_copy(table_hbm.at[idxbuf.at[slot, pl.ds(0, 56)]],
                             rows.at[slot], sem)

        def wait_slot(slot, sem):
            pltpu.make_async_copy(table_hbm.at[idxbuf.at[slot, pl.ds(0, 56)]],
                                  rows.at[slot], sem).wait()

        def accum(gy, slot, even):
            bin_base = (gy // 2) * (OUT * 256) + par_off
            gyv = jnp.full((16,), gy, jnp.int32)
            w0 = wy0.at[gyv].get(mode="promise_in_bounds")
            w1 = wy1.at[gyv].get(mode="promise_in_bounds")
            wv = (wx0 * w0, wx1 * w0, wx0 * w1, wx1 * w1)

            @plsc.parallel_loop(0, OUT)
            def _px(px):
                l0 = jnp.full((16,), 2 * px, jnp.int32)
                l1 = l0 + 1
                wb = []
                for v in range(4):
                    wb.append(wv[v].at[l0].get(mode="promise_in_bounds"))
                    wb.append(wv[v].at[l1].get(mode="promise_in_bounds"))
                off = bin_base + px * 256
                roff = 8 * px
                for g in range(8):
                    sl = pl.ds(g * 16, 16)
                    s0 = s1 = None
                    for v in range(4):
                        for k, dj in ((2 * v, v), (2 * v + 1, 4 + v)):
                            a, c = plsc.unpack(
                                plsc.bitcast(rows[slot, roff + dj, sl],
                                             jnp.bfloat16),
                                format=plsc.PackFormat.INTERLEAVED,
                                preferred_element_type=jnp.float32)
                            w = wb[k]
                            s0 = w * a if s0 is None else s0 + w * a
                            s1 = w * c if s1 is None else s1 + w * c
                    offg = off + g * 32
                    if even:
                        acc[pl.ds(offg, 16)] = s0
                        acc[pl.ds(offg + 16, 16)] = s1
                    else:
                        plsc.addupdate(acc.at[pl.ds(offg, 16)], s0)
                        plsc.addupdate(acc.at[pl.ds(offg + 16, 16)], s1)

        build_start(0, 0, sems[0])
        build_start(1, 1, sems[1])
        build_start(2, 2, sems[2])

        @pl.loop(0, GP, step=4)
        def _quad(gy):
            @pl.when(gy + 3 < G)
            def _():
                build_start(gy + 3, 3, sems[3])

            wait_slot(0, sems[0])
            accum(gy, 0, True)

            @pl.when(gy + 4 < G)
            def _():
                build_start(gy + 4, 0, sems[0])

            wait_slot(1, sems[1])
            accum(gy + 1, 1, False)

            @pl.when(gy + 5 < G)
            def _():
                build_start(gy + 5, 1, sems[1])

            @pl.when(gy + 2 < G)
            def _():
                wait_slot(2, sems[2])
                accum(gy + 2, 2, True)

            @pl.when(gy + 6 < G)
            def _():
                build_start(gy + 6, 2, sems[2])

            @pl.when(gy + 3 < G)
            def _():
                wait_slot(3, sems[3])
                accum(gy + 3, 3, False)

        @pl.when(par_even)
        def _():
            pltpu.async_copy(acc.at[pl.ds(0, ACC)],
                             out_hbm.at[base_box + b], semo0)

        @pl.when(jnp.logical_not(par_even))
        def _():
            pltpu.async_copy(acc.at[pl.ds(ACCP, ACC)],
                             out_hbm.at[base_box + b], semo1)

    pltpu.make_async_copy(acc.at[pl.ds(0, ACC)],
                          out_hbm.at[base_box], semo0).wait()
    pltpu.make_async_copy(acc.at[pl.ds(ACCP, ACC)],
                          out_hbm.at[base_box], semo1).wait()


@jax.jit
def _run(table, boxesT):
    mesh = plsc.VectorSubcoreMesh(core_axis_name="c", subcore_axis_name="s")
    cp = pltpu.CompilerParams()
    if "needs_layout_passes" in pltpu.CompilerParams.__dataclass_fields__:
        cp = dataclasses.replace(cp, needs_layout_passes=False)
    f = pl.kernel(
        _sc_kernel,
        out_type=jax.ShapeDtypeStruct((KPAD, ACC), jnp.float32),
        mesh=mesh,
        compiler_params=cp,
        scratch_types=[
            pltpu.VMEM((4, BPW), jnp.float32),
            pltpu.VMEM((BPW + 16,), jnp.float32),
            pltpu.VMEM((BPW + 16,), jnp.float32),
            pltpu.VMEM((BPW + 16,), jnp.float32),
            pltpu.VMEM((BPW + 16,), jnp.float32),
            pltpu.VMEM((BPW + 16,), jnp.int32),
            pltpu.VMEM((BPW + 16,), jnp.int32),
            pltpu.VMEM((4, 64), jnp.int32),
            pltpu.VMEM((4, 56, 128), jnp.int32),
            pltpu.VMEM((2 * ACCP,), jnp.float32),
            pltpu.SemaphoreType.DMA,
            pltpu.SemaphoreType.DMA,
            pltpu.SemaphoreType.DMA,
            pltpu.SemaphoreType.DMA,
            pltpu.SemaphoreType.DMA,
            pltpu.SemaphoreType.DMA,
        ],
    )
    return f(table, boxesT)


def kernel(feat0, feat1, feat2, feat3, boxes):
    table = jnp.concatenate(
        [jnp.transpose(f[0, _PERM].reshape(256, -1))
         for f in (feat0, feat1, feat2, feat3)],
        axis=0).astype(jnp.bfloat16)
    table = lax.bitcast_convert_type(table.reshape(_NROWS, 128, 2), jnp.int32)
    k = boxes.shape[0]
    boxesT = jnp.transpose(jnp.pad(boxes, ((0, KPAD - k), (0, 0))))
    out = _run(table, boxesT)
    out = out.reshape(KPAD, OUT, OUT, 256)[:k]
    return jnp.transpose(out, (0, 3, 1, 2))


# speedup vs baseline: 1.4636x; 1.4636x over previous
import functools

import jax
import jax.numpy as jnp
from jax import lax
from jax.experimental import pallas as pl
from jax.experimental.pallas import tpu as pltpu
from jax.experimental.pallas import tpu_sc as plsc

OUT = 7
SR = 2
G = OUT * SR
IMG = 1024.0
EPS = 1e-6

_SIZES = (256, 128, 64, 32)
_BASES = (0, 65536, 81920, 86016)
_NROWS = 87040

_T1 = (224.0 * 0.5) ** 2 * 2.0 ** (-2 * EPS)
_T2 = 224.0**2 * 2.0 ** (-2 * EPS)
_T3 = (224.0 * 2.0) ** 2 * 2.0 ** (-2 * EPS)

NC = 2
NS = 16
NW = NC * NS
KPAD = 1024
BPW = KPAD // NW
ACC = OUT * OUT * 256


def _sc_kernel(table_hbm, boxesT_hbm, out_hbm,
               boxes_v, x1s, y1s, bws, bhs, basei, wfi,
               idxbuf, rows, acc,
               sem0, sem1, sem2, sem3, semo0, semo1):
    wid = lax.axis_index("s") * NC + lax.axis_index("c")
    base_box = wid * BPW

    for i in range(4):
        pltpu.sync_copy(boxesT_hbm.at[i, pl.ds(base_box, BPW)], boxes_v.at[i])

    for g in range(BPW // 16):
        sl = pl.ds(g * 16, 16)
        x1 = boxes_v[0, sl]
        y1 = boxes_v[1, sl]
        x2 = boxes_v[2, sl]
        y2 = boxes_v[3, sl]
        area = (x2 - x1) * (y2 - y1)
        i32 = jnp.int32
        one = jnp.ones((16,), i32)
        zero = jnp.zeros((16,), i32)
        cnt = (jnp.where(area >= _T1, one, zero)
               + jnp.where(area >= _T2, one, zero)
               + jnp.where(area >= _T3, one, zero))
        scale = jnp.where(cnt == 0, 0.25,
                          jnp.where(cnt == 1, 0.125,
                                    jnp.where(cnt == 2, 0.0625, 0.03125)))
        wi = jnp.where(cnt == 0, _SIZES[0],
                       jnp.where(cnt == 1, _SIZES[1],
                                 jnp.where(cnt == 2, _SIZES[2], _SIZES[3])))
        bi = jnp.where(cnt == 0, _BASES[0],
                       jnp.where(cnt == 1, _BASES[1],
                                 jnp.where(cnt == 2, _BASES[2], _BASES[3])))
        x1f = x1 * scale
        y1f = y1 * scale
        rw = jnp.maximum(x2 * scale - x1f, 1.0)
        rh = jnp.maximum(y2 * scale - y1f, 1.0)
        x1s[sl] = x1f
        y1s[sl] = y1f
        bws[sl] = rw / float(OUT)
        bhs[sl] = rh / float(OUT)
        basei[sl] = bi.astype(i32)
        wfi[sl] = wi.astype(i32)

    lane = lax.iota(jnp.int32, 16)
    t = (lane.astype(jnp.float32) + 0.5) * (1.0 / SR)
    act = lane < G

    sems = (sem0, sem1, sem2, sem3)
    ioq = lane >> 2
    dymask = ((lane >> 1) & 1) == 1
    dxv = lane & 1

    @pl.loop(0, BPW)
    def _box(b):
        par_even = (b % 2) == 0
        par_off = (b % 2) * ACC

        @pl.when(jnp.logical_and(b >= 2, par_even))
        def _():
            pltpu.make_async_copy(acc.at[pl.ds(0, ACC)],
                                  out_hbm.at[base_box + b], semo0).wait()

        @pl.when(jnp.logical_and(b >= 2, jnp.logical_not(par_even)))
        def _():
            pltpu.make_async_copy(acc.at[pl.ds(ACC, ACC)],
                                  out_hbm.at[base_box + b], semo1).wait()

        x1b = x1s[pl.ds(b, 16)][0]
        y1b = y1s[pl.ds(b, 16)][0]
        bwb = bws[pl.ds(b, 16)][0]
        bhb = bhs[pl.ds(b, 16)][0]
        bb = basei[pl.ds(b, 16)][0]
        wib = wfi[pl.ds(b, 16)][0]
        wfb = wib.astype(jnp.float32)

        vx = jnp.clip(x1b + t * bwb, 0.0, wfb - 1.0)
        xli = jnp.minimum(vx.astype(jnp.int32), wib - 2)
        fx = vx - xli.astype(jnp.float32)
        wx0 = jnp.where(act, (1.0 - fx) * 0.25, 0.0)
        wx1 = jnp.where(act, fx * 0.25, 0.0)

        vy = jnp.clip(y1b + t * bhb, 0.0, wfb - 1.0)
        yli = jnp.minimum(vy.astype(jnp.int32), wib - 2)
        fy = vy - yli.astype(jnp.float32)
        row0 = bb + yli * wib
        row1 = row0 + wib
        wy0 = 1.0 - fy
        wy1 = fy

        def build_start(gy, slot, sem):
            gyv = jnp.full((16,), gy, jnp.int32)
            r0 = row0.at[gyv].get(mode="promise_in_bounds")
            r1 = row1.at[gyv].get(mode="promise_in_bounds")
            rsel = jnp.where(dymask, r1, r0) + dxv
            for k in range(4):
                gxk = ioq + 4 * k
                xk = xli.at[gxk].get(mode="promise_in_bounds")
                idxbuf[slot, pl.ds(16 * k, 16)] = xk + rsel
            pltpu.async_copy(table_hbm.at[idxbuf.at[slot, pl.ds(0, 56)]],
                             rows.at[slot], sem)

        def wait_slot(slot, sem):
            pltpu.make_async_copy(table_hbm.at[idxbuf.at[slot, pl.ds(0, 56)]],
                                  rows.at[slot], sem).wait()

        def accum(gy, slot, even):
            bin_base = (gy // 2) * (OUT * 256) + par_off
            gyv = jnp.full((16,), gy, jnp.int32)
            w0 = wy0.at[gyv].get(mode="promise_in_bounds")
            w1 = wy1.at[gyv].get(mode="promise_in_bounds")
            wv = (wx0 * w0, wx1 * w0, wx0 * w1, wx1 * w1)

            @plsc.parallel_loop(0, OUT)
            def _px(px):
                l0 = jnp.full((16,), 2 * px, jnp.int32)
                l1 = l0 + 1
                wb = []
                for v in range(4):
                    wb.append(wv[v].at[l0].get(mode="promise_in_bounds"))
                    wb.append(wv[v].at[l1].get(mode="promise_in_bounds"))
                off = bin_base + px * 256
                roff = 8 * px
                for ci in range(16):
                    sl = pl.ds(ci * 16, 16)
                    s = None
                    for v in range(4):
                        term = (wb[2 * v] * rows[slot, roff + v, sl]
                                + wb[2 * v + 1] * rows[slot, roff + 4 + v, sl])
                        s = term if s is None else s + term
                    if even:
                        acc[pl.ds(off + ci * 16, 16)] = s
                    else:
                        plsc.addupdate(acc.at[pl.ds(off + ci * 16, 16)], s)

        build_start(0, 0, sems[0])
        build_start(1, 1, sems[1])

        @pl.loop(0, 6, step=2)
        def _quad(q):
            gy = 2 * q
            build_start(gy + 2, 2, sems[2])
            wait_slot(0, sems[0])
            accum(gy, 0, True)
            build_start(gy + 3, 3, sems[3])
            wait_slot(1, sems[1])
            accum(gy + 1, 1, False)
            build_start(gy + 4, 0, sems[0])
            wait_slot(2, sems[2])
            accum(gy + 2, 2, True)
            build_start(gy + 5, 1, sems[1])
            wait_slot(3, sems[3])
            accum(gy + 3, 3, False)

        wait_slot(0, sems[0])
        accum(G - 2, 0, True)
        wait_slot(1, sems[1])
        accum(G - 1, 1, False)

        @pl.when(par_even)
        def _():
            pltpu.async_copy(acc.at[pl.ds(0, ACC)],
                             out_hbm.at[base_box + b], semo0)

        @pl.when(jnp.logical_not(par_even))
        def _():
            pltpu.async_copy(acc.at[pl.ds(ACC, ACC)],
                             out_hbm.at[base_box + b], semo1)

    pltpu.make_async_copy(acc.at[pl.ds(0, ACC)],
                          out_hbm.at[base_box], semo0).wait()
    pltpu.make_async_copy(acc.at[pl.ds(ACC, ACC)],
                          out_hbm.at[base_box], semo1).wait()


@jax.jit
def _run(table, boxesT):
    mesh = plsc.VectorSubcoreMesh(core_axis_name="c", subcore_axis_name="s")
    f = pl.kernel(
        _sc_kernel,
        out_type=jax.ShapeDtypeStruct((KPAD, ACC), jnp.float32),
        mesh=mesh,
        scratch_types=[
            pltpu.VMEM((4, BPW), jnp.float32),
            pltpu.VMEM((BPW + 16,), jnp.float32),
            pltpu.VMEM((BPW + 16,), jnp.float32),
            pltpu.VMEM((BPW + 16,), jnp.float32),
            pltpu.VMEM((BPW + 16,), jnp.float32),
            pltpu.VMEM((BPW + 16,), jnp.int32),
            pltpu.VMEM((BPW + 16,), jnp.int32),
            pltpu.VMEM((4, 64), jnp.int32),
            pltpu.VMEM((4, 56, 256), jnp.float32),
            pltpu.VMEM((2 * ACC,), jnp.float32),
            pltpu.SemaphoreType.DMA,
            pltpu.SemaphoreType.DMA,
            pltpu.SemaphoreType.DMA,
            pltpu.SemaphoreType.DMA,
            pltpu.SemaphoreType.DMA,
            pltpu.SemaphoreType.DMA,
        ],
    )
    return f(table, boxesT)


def kernel(feat0, feat1, feat2, feat3, boxes):
    table = jnp.concatenate(
        [jnp.transpose(f[0].reshape(256, -1))
         for f in (feat0, feat1, feat2, feat3)],
        axis=0)
    k = boxes.shape[0]
    boxesT = jnp.transpose(jnp.pad(boxes, ((0, KPAD - k), (0, 0))))
    out = _run(table, boxesT)
    out = out.reshape(KPAD, OUT, OUT, 256)[:k]
    return jnp.transpose(out, (0, 3, 1, 2))


# speedup vs baseline: 1.4668x; 1.0022x over previous
import jax
import jax.numpy as jnp
from jax import lax
from jax.experimental import pallas as pl
from jax.experimental.pallas import tpu as pltpu
from jax.experimental.pallas import tpu_sc as plsc

OUT = 7
SR = 2
G = OUT * SR
IMG = 1024.0
EPS = 1e-6

_SIZES = (256, 128, 64, 32)
_BASES = (0, 65536, 81920, 86016)
_NROWS = 87040

_T1 = (224.0 * 0.5) ** 2 * 2.0 ** (-2 * EPS)
_T2 = 224.0**2 * 2.0 ** (-2 * EPS)
_T3 = (224.0 * 2.0) ** 2 * 2.0 ** (-2 * EPS)

NC = 2
NS = 16
NW = NC * NS
KPAD = 1024
BPW = KPAD // NW
ACC = OUT * OUT * 256


def _sc_kernel(table_hbm, boxesT_hbm, out_hbm,
               boxes_v, x1s, y1s, bws, bhs, basei, wfi,
               idxbuf, rows, acc,
               sem0, sem1, sem2, sem3, semo0, semo1):
    wid = lax.axis_index("s") * NC + lax.axis_index("c")
    base_box = wid * BPW

    for i in range(4):
        pltpu.sync_copy(boxesT_hbm.at[i, pl.ds(base_box, BPW)], boxes_v.at[i])

    for g in range(BPW // 16):
        sl = pl.ds(g * 16, 16)
        x1 = boxes_v[0, sl]
        y1 = boxes_v[1, sl]
        x2 = boxes_v[2, sl]
        y2 = boxes_v[3, sl]
        area = (x2 - x1) * (y2 - y1)
        i32 = jnp.int32
        one = jnp.ones((16,), i32)
        zero = jnp.zeros((16,), i32)
        cnt = (jnp.where(area >= _T1, one, zero)
               + jnp.where(area >= _T2, one, zero)
               + jnp.where(area >= _T3, one, zero))
        scale = jnp.where(cnt == 0, 0.25,
                          jnp.where(cnt == 1, 0.125,
                                    jnp.where(cnt == 2, 0.0625, 0.03125)))
        wi = jnp.where(cnt == 0, _SIZES[0],
                       jnp.where(cnt == 1, _SIZES[1],
                                 jnp.where(cnt == 2, _SIZES[2], _SIZES[3])))
        bi = jnp.where(cnt == 0, _BASES[0],
                       jnp.where(cnt == 1, _BASES[1],
                                 jnp.where(cnt == 2, _BASES[2], _BASES[3])))
        x1f = x1 * scale
        y1f = y1 * scale
        rw = jnp.maximum(x2 * scale - x1f, 1.0)
        rh = jnp.maximum(y2 * scale - y1f, 1.0)
        x1s[sl] = x1f
        y1s[sl] = y1f
        bws[sl] = rw / float(OUT)
        bhs[sl] = rh / float(OUT)
        basei[sl] = bi.astype(i32)
        wfi[sl] = wi.astype(i32)

    lane = lax.iota(jnp.int32, 16)
    t = (lane.astype(jnp.float32) + 0.5) * (1.0 / SR)
    act = lane < G

    sems = (sem0, sem1, sem2, sem3)
    ioq = lane >> 2
    dymask = ((lane >> 1) & 1) == 1
    dxv = lane & 1

    @pl.loop(0, BPW)
    def _box(b):
        par_even = (b % 2) == 0
        par_off = (b % 2) * ACC

        @pl.when(jnp.logical_and(b >= 2, par_even))
        def _():
            pltpu.make_async_copy(acc.at[pl.ds(0, ACC)],
                                  out_hbm.at[base_box + b], semo0).wait()

        @pl.when(jnp.logical_and(b >= 2, jnp.logical_not(par_even)))
        def _():
            pltpu.make_async_copy(acc.at[pl.ds(ACC, ACC)],
                                  out_hbm.at[base_box + b], semo1).wait()

        x1b = x1s[pl.ds(b, 16)][0]
        y1b = y1s[pl.ds(b, 16)][0]
        bwb = bws[pl.ds(b, 16)][0]
        bhb = bhs[pl.ds(b, 16)][0]
        bb = basei[pl.ds(b, 16)][0]
        wib = wfi[pl.ds(b, 16)][0]
        wfb = wib.astype(jnp.float32)

        vx = jnp.clip(x1b + t * bwb, 0.0, wfb - 1.0)
        xli = jnp.minimum(vx.astype(jnp.int32), wib - 2)
        fx = vx - xli.astype(jnp.float32)
        wx0 = jnp.where(act, (1.0 - fx) * 0.25, 0.0)
        wx1 = jnp.where(act, fx * 0.25, 0.0)

        vy = jnp.clip(y1b + t * bhb, 0.0, wfb - 1.0)
        yli = jnp.minimum(vy.astype(jnp.int32), wib - 2)
        fy = vy - yli.astype(jnp.float32)
        row0 = bb + yli * wib
        row1 = row0 + wib
        wy0 = 1.0 - fy
        wy1 = fy

        def build_start(gy, slot, sem):
            gyv = jnp.full((16,), gy, jnp.int32)
            r0 = row0.at[gyv].get(mode="promise_in_bounds")
            r1 = row1.at[gyv].get(mode="promise_in_bounds")
            rsel = jnp.where(dymask, r1, r0) + dxv
            for k in range(4):
                gxk = ioq + 4 * k
                xk = xli.at[gxk].get(mode="promise_in_bounds")
                idxbuf[slot, pl.ds(16 * k, 16)] = xk + rsel
            pltpu.async_copy(table_hbm.at[idxbuf.at[slot, pl.ds(0, 56)]],
                             rows.at[slot], sem)

        def wait_slot(slot, sem):
            pltpu.make_async_copy(table_hbm.at[idxbuf.at[slot, pl.ds(0, 56)]],
                                  rows.at[slot], sem).wait()

        def accum(gy, slot, even):
            bin_base = (gy // 2) * (OUT * 256) + par_off
            gyv = jnp.full((16,), gy, jnp.int32)
            w0 = wy0.at[gyv].get(mode="promise_in_bounds")
            w1 = wy1.at[gyv].get(mode="promise_in_bounds")
            wv = (wx0 * w0, wx1 * w0, wx0 * w1, wx1 * w1)

            @plsc.parallel_loop(0, OUT)
            def _px(px):
                l0 = jnp.full((16,), 2 * px, jnp.int32)
                l1 = l0 + 1
                wb = []
                for v in range(4):
                    wb.append(wv[v].at[l0].get(mode="promise_in_bounds"))
                    wb.append(wv[v].at[l1].get(mode="promise_in_bounds"))
                off = bin_base + px * 256
                roff = 8 * px
                for ci in range(16):
                    sl = pl.ds(ci * 16, 16)
                    s = None
                    for v in range(4):
                        term = (wb[2 * v] * rows[slot, roff + v, sl]
                                + wb[2 * v + 1] * rows[slot, roff + 4 + v, sl])
                        s = term if s is None else s + term
                    if even:
                        acc[pl.ds(off + ci * 16, 16)] = s
                    else:
                        plsc.addupdate(acc.at[pl.ds(off + ci * 16, 16)], s)

        build_start(0, 0, sems[0])
        build_start(1, 1, sems[1])

        @pl.loop(0, 6, step=2)
        def _quad(q):
            gy = 2 * q
            build_start(gy + 2, 2, sems[2])
            wait_slot(0, sems[0])
            accum(gy, 0, True)
            build_start(gy + 3, 3, sems[3])
            wait_slot(1, sems[1])
            accum(gy + 1, 1, False)
            build_start(gy + 4, 0, sems[0])
            wait_slot(2, sems[2])
            accum(gy + 2, 2, True)
            build_start(gy + 5, 1, sems[1])
            wait_slot(3, sems[3])
            accum(gy + 3, 3, False)

        wait_slot(0, sems[0])
        accum(G - 2, 0, True)
        wait_slot(1, sems[1])
        accum(G - 1, 1, False)

        @pl.when(par_even)
        def _():
            pltpu.async_copy(acc.at[pl.ds(0, ACC)],
                             out_hbm.at[base_box + b], semo0)

        @pl.when(jnp.logical_not(par_even))
        def _():
            pltpu.async_copy(acc.at[pl.ds(ACC, ACC)],
                             out_hbm.at[base_box + b], semo1)

    pltpu.make_async_copy(acc.at[pl.ds(0, ACC)],
                          out_hbm.at[base_box], semo0).wait()
    pltpu.make_async_copy(acc.at[pl.ds(ACC, ACC)],
                          out_hbm.at[base_box], semo1).wait()


@jax.jit
def _run(table, boxesT):
    mesh = plsc.VectorSubcoreMesh(core_axis_name="c", subcore_axis_name="s")
    f = pl.kernel(
        _sc_kernel,
        out_type=jax.ShapeDtypeStruct((KPAD, ACC), jnp.float32),
        mesh=mesh,
        scratch_types=[
            pltpu.VMEM((4, BPW), jnp.float32),
            pltpu.VMEM((BPW + 16,), jnp.float32),
            pltpu.VMEM((BPW + 16,), jnp.float32),
            pltpu.VMEM((BPW + 16,), jnp.float32),
            pltpu.VMEM((BPW + 16,), jnp.float32),
            pltpu.VMEM((BPW + 16,), jnp.int32),
            pltpu.VMEM((BPW + 16,), jnp.int32),
            pltpu.VMEM((4, 64), jnp.int32),
            pltpu.VMEM((4, 56, 256), jnp.float32),
            pltpu.VMEM((2 * ACC,), jnp.float32),
            pltpu.SemaphoreType.DMA,
            pltpu.SemaphoreType.DMA,
            pltpu.SemaphoreType.DMA,
            pltpu.SemaphoreType.DMA,
            pltpu.SemaphoreType.DMA,
            pltpu.SemaphoreType.DMA,
        ],
    )
    return f(table, boxesT)


def kernel(feat0, feat1, feat2, feat3, boxes):
    table = jnp.concatenate(
        [jnp.transpose(f[0].reshape(256, -1))
         for f in (feat0, feat1, feat2, feat3)],
        axis=0)
    k = boxes.shape[0]
    boxesT = jnp.transpose(jnp.pad(boxes, ((0, KPAD - k), (0, 0))))
    out = _run(table, boxesT)
    out = out.reshape(KPAD, OUT, OUT, 256)[:k]
    return jnp.transpose(out, (0, 3, 1, 2))
